# Initial kernel scaffold; baseline (speedup 1.0000x reference)
import jax
import jax.numpy as jnp
from jax.experimental import pallas as pl


def kernel(x, edge_index, batch, W1, b1, W2, b2, fc_W, fc_b):
    raise NotImplementedError("write your pallas kernel here")


# speedup vs baseline: 13.5145x; 13.5145x over previous
import functools

import jax
import jax.numpy as jnp
from jax import lax
from jax.experimental import pallas as pl
from jax.experimental.pallas import tpu as pltpu
from jax.experimental.pallas import tpu_sc as plsc

N_NODES = 10000
N_EDGES = 320000
N_GRAPHS = 64
D_IN = 128
D_HID = 16

NC, NS = 2, 16
NW = NC * NS
R = 10112
RS = 10240
TRASH = R
CH = 128
NCH = 79
E_PAD = NW * NCH * CH
ZROWS = RS // NS
OROWS = R // NS

_mesh = plsc.VectorSubcoreMesh(core_axis_name="c", subcore_axis_name="s")


def _sc_agg_body(with_deg, *refs):
    if with_deg:
        (h_hbm, src_hbm, dst_hbm, zeros_hbm, ones_hbm, p_hbm, dg_hbm,
         src_v, dst_v, rows_v, ones_v, agg_s, deg_s, sem) = refs
    else:
        (h_hbm, src_hbm, dst_hbm, zeros_hbm, p_hbm,
         src_v, dst_v, rows_v, agg_s, sem) = refs
    c = lax.axis_index("c")
    s = lax.axis_index("s")
    wid = c * NS + s
    pltpu.sync_copy(zeros_hbm, agg_s.at[pl.ds(s * ZROWS, ZROWS)])
    if with_deg:
        pltpu.sync_copy(zeros_hbm, deg_s.at[pl.ds(s * ZROWS, ZROWS)])
        pltpu.sync_copy(ones_hbm, ones_v)
    pltpu.sync_copy(src_hbm.at[wid], src_v)
    pltpu.sync_copy(dst_hbm.at[wid], dst_v)
    plsc.subcore_barrier()

    def body(j, carry):
        pltpu.async_copy(h_hbm.at[src_v.at[j]], rows_v, sem).wait()
        pltpu.sync_copy(rows_v, agg_s.at[dst_v.at[j]], add=True)
        if with_deg:
            pltpu.sync_copy(ones_v, deg_s.at[dst_v.at[j]], add=True)
        return carry

    lax.fori_loop(0, NCH, body, 0)
    plsc.subcore_barrier()
    pltpu.sync_copy(agg_s.at[pl.ds(s * OROWS, OROWS)],
                    p_hbm.at[c, pl.ds(s * OROWS, OROWS)])
    if with_deg:
        pltpu.sync_copy(deg_s.at[pl.ds(s * OROWS, OROWS)],
                        dg_hbm.at[c, pl.ds(s * OROWS, OROWS)])


_sc_params = pltpu.CompilerParams(use_tc_tiling_on_sc=False)

_sc_agg_deg = functools.partial(
    pl.kernel,
    mesh=_mesh,
    compiler_params=_sc_params,
    out_type=[jax.ShapeDtypeStruct((NC, R, D_HID), jnp.float32),
              jax.ShapeDtypeStruct((NC, R, D_HID), jnp.float32)],
    scratch_types=[
        pltpu.VMEM((NCH, CH), jnp.int32),
        pltpu.VMEM((NCH, CH), jnp.int32),
        pltpu.VMEM((CH, D_HID), jnp.float32),
        pltpu.VMEM((CH, D_HID), jnp.float32),
        pltpu.VMEM_SHARED((RS, D_HID), jnp.float32),
        pltpu.VMEM_SHARED((RS, D_HID), jnp.float32),
        pltpu.SemaphoreType.DMA,
    ],
)(functools.partial(_sc_agg_body, True))

_sc_agg = functools.partial(
    pl.kernel,
    mesh=_mesh,
    compiler_params=_sc_params,
    out_type=[jax.ShapeDtypeStruct((NC, R, D_HID), jnp.float32)],
    scratch_types=[
        pltpu.VMEM((NCH, CH), jnp.int32),
        pltpu.VMEM((NCH, CH), jnp.int32),
        pltpu.VMEM((CH, D_HID), jnp.float32),
        pltpu.VMEM_SHARED((RS, D_HID), jnp.float32),
        pltpu.SemaphoreType.DMA,
    ],
)(functools.partial(_sc_agg_body, False))


def _mm1_body(x_ref, w_ref, o_ref):
    o_ref[...] = jnp.dot(x_ref[...], w_ref[...],
                         preferred_element_type=jnp.float32)


def _mid_body(p_ref, d_ref, b1_ref, w2_ref, o_ref):
    agg = p_ref[0] + p_ref[1]
    deg = jnp.maximum(d_ref[0] + d_ref[1], 1.0)
    h1 = jnp.maximum(agg / deg + b1_ref[...], 0.0)
    o_ref[...] = jnp.dot(h1, w2_ref[...], preferred_element_type=jnp.float32)


def _final_body(p_ref, d_ref, b2_ref, batch_ref, fcw_ref, fcb_ref, o_ref):
    agg = p_ref[0] + p_ref[1]
    deg = jnp.maximum(d_ref[0] + d_ref[1], 1.0)
    h2 = jnp.maximum(agg / deg + b2_ref[...], 0.0)
    b = batch_ref[...]
    gid = lax.broadcasted_iota(jnp.int32, (N_GRAPHS, R), 0)
    onehot = (b == gid).astype(jnp.float32)
    sums = jnp.dot(onehot, h2, preferred_element_type=jnp.float32)
    counts = jnp.sum(onehot, axis=1, keepdims=True)
    pooled = sums / jnp.maximum(counts, 1.0)
    o_ref[...] = (jnp.dot(pooled, fcw_ref[...],
                          preferred_element_type=jnp.float32) + fcb_ref[...])


_mm1 = pl.pallas_call(
    _mm1_body, out_shape=jax.ShapeDtypeStruct((R, D_HID), jnp.float32))

_mid = pl.pallas_call(
    _mid_body, out_shape=jax.ShapeDtypeStruct((R, D_HID), jnp.float32))

_final = pl.pallas_call(
    _final_body, out_shape=jax.ShapeDtypeStruct((N_GRAPHS, 1), jnp.float32))


def kernel(x, edge_index, batch, W1, b1, W2, b2, fc_W, fc_b):
    src = edge_index[0].astype(jnp.int32)
    dst = edge_index[1].astype(jnp.int32)
    n_pad_e = E_PAD - N_EDGES
    src3 = jnp.concatenate(
        [src, jnp.zeros((n_pad_e,), jnp.int32)]).reshape(NW, NCH, CH)
    dst3 = jnp.concatenate(
        [dst, jnp.full((n_pad_e,), TRASH, jnp.int32)]).reshape(NW, NCH, CH)
    x_pad = jnp.pad(x, ((0, R - N_NODES), (0, 0)))
    batch2 = jnp.concatenate(
        [batch.astype(jnp.int32),
         jnp.full((R - N_NODES,), N_GRAPHS, jnp.int32)]).reshape(1, R)
    zeros_z = jnp.zeros((ZROWS, D_HID), jnp.float32)
    ones_ch = jnp.ones((CH, D_HID), jnp.float32)

    h0 = _mm1(x_pad, W1)
    p1, dg = _sc_agg_deg(h0, src3, dst3, zeros_z, ones_ch)
    h1b = _mid(p1, dg, b1.reshape(1, D_HID), W2)
    (p2,) = _sc_agg(h1b, src3, dst3, zeros_z)
    out = _final(p2, dg, b2.reshape(1, D_HID), batch2, fc_W,
                 fc_b.reshape(1, 1))
    return out
